# baseline (speedup 1.0000x reference)
import functools

import jax
import jax.numpy as jnp
from jax import lax
from jax.experimental import pallas as pl
from jax.experimental.pallas import tpu as pltpu
from jax.experimental.pallas import tpu_sc as plsc

EMBED_DIM = 64
HIDDEN_DIM = 256
IDX_CHUNK = 128


def _transpose_body(tt_ref, out_ref):
    z = tt_ref[...].T
    half = z.shape[0] // 2
    out_ref[...] = jnp.concatenate([z[:half], z[half:]], axis=1)


@functools.cache
def _transpose_fn(N: int, D: int, blk: int):
    grid = (N + blk - 1) // blk
    return pl.pallas_call(
        _transpose_body,
        grid=(grid,),
        in_specs=[
            pl.BlockSpec((D, blk), lambda i: (0, i)),
        ],
        out_specs=pl.BlockSpec((blk // 2, 2 * D), lambda i: (i, 0)),
        out_shape=jax.ShapeDtypeStruct((grid * (blk // 2), 2 * D), jnp.float32),
        compiler_params=pltpu.CompilerParams(vmem_limit_bytes=100 * 2**20),
    )


@functools.cache
def _gather_fn(B: int, N2: int, W: int):
    info = plsc.get_sparse_core_info()
    NC, NS = info.num_cores, info.num_subcores
    NW = NC * NS
    assert B % (NW * IDX_CHUNK) == 0
    n_chunks = B // (NW * IDX_CHUNK)
    b_per_w = n_chunks * IDX_CHUNK
    mesh = plsc.VectorSubcoreMesh(core_axis_name="c", subcore_axis_name="s")

    @functools.partial(
        pl.kernel,
        mesh=mesh,
        out_type=jax.ShapeDtypeStruct((B, W), jnp.float32),
        scratch_types=[
            pltpu.VMEM((n_chunks, IDX_CHUNK), jnp.int32),
            pltpu.VMEM((b_per_w, W), jnp.float32),
            pltpu.SemaphoreType.DMA,
        ],
    )
    def gather(idx_hbm, table_hbm, out_hbm, idx_v, rows_v, sem):
        wid = lax.axis_index("s") * NC + lax.axis_index("c")
        base = wid * b_per_w
        pltpu.sync_copy(idx_hbm.at[wid], idx_v)
        copies = []
        for c in range(n_chunks):
            copies.append(
                pltpu.async_copy(
                    table_hbm.at[idx_v.at[c]],
                    rows_v.at[pl.ds(c * IDX_CHUNK, IDX_CHUNK)],
                    sem,
                )
            )
        for cp in copies:
            cp.wait()
        pltpu.sync_copy(rows_v, out_hbm.at[pl.ds(base, b_per_w)])

    def run(idx_half, table2):
        idx3 = idx_half.reshape(NW, n_chunks, IDX_CHUNK)
        return gather(idx3, table2)

    return run


def _mlp_body(packed_ref, half_ref, w1_ref, b1_ref, w2_ref, b2_ref, out_ref):
    par = half_ref[...].astype(jnp.float32)
    p0 = packed_ref[:, :EMBED_DIM]
    p1 = packed_ref[:, EMBED_DIM:]
    emb = p0 * (1.0 - par) + p1 * par
    h = jnp.dot(emb, w1_ref[...], preferred_element_type=jnp.float32)
    h = jnp.maximum(h + b1_ref[...], 0.0)
    out_ref[...] = (
        jnp.dot(h, w2_ref[...], preferred_element_type=jnp.float32) + b2_ref[...]
    )


@functools.cache
def _mlp_fn(B: int, D: int, H: int, blk: int):
    grid = B // blk
    return pl.pallas_call(
        _mlp_body,
        grid=(grid,),
        in_specs=[
            pl.BlockSpec((blk, 2 * D), lambda i: (i, 0)),
            pl.BlockSpec((blk, 1), lambda i: (i, 0)),
            pl.BlockSpec((D, H), lambda i: (0, 0)),
            pl.BlockSpec((1, H), lambda i: (0, 0)),
            pl.BlockSpec((H, D), lambda i: (0, 0)),
            pl.BlockSpec((1, D), lambda i: (0, 0)),
        ],
        out_specs=pl.BlockSpec((blk, D), lambda i: (i, 0)),
        out_shape=jax.ShapeDtypeStruct((B, D), jnp.float32),
    )


def kernel(user_id, table, W1, b1, W2, b2):
    B = user_id.shape[0]
    N, D = table.shape
    H = W1.shape[1]
    blk = 40960
    idx = user_id.reshape(B).astype(jnp.int32)
    table2 = _transpose_fn(N, D, blk)(table.T)
    blk_i = idx // blk
    off = idx % blk
    packed_row = blk_i * (blk // 2) + off % (blk // 2)
    half = off // (blk // 2)
    packed = _gather_fn(B, table2.shape[0], 2 * D)(packed_row, table2)
    out = _mlp_fn(B, D, H, 2048)(
        packed,
        half.reshape(B, 1),
        W1,
        b1.reshape(1, H),
        W2,
        b2.reshape(1, D),
    )
    return out


# speedup vs baseline: 1.0692x; 1.0692x over previous
import functools

import jax
import jax.numpy as jnp
from jax import lax
from jax.experimental import pallas as pl
from jax.experimental.pallas import tpu as pltpu
from jax.experimental.pallas import tpu_sc as plsc

EMBED_DIM = 64
HIDDEN_DIM = 256
IDX_CHUNK = 128


def _transpose_body(tt_ref, out_ref):
    z = tt_ref[...].T
    half = z.shape[0] // 2
    out_ref[...] = jnp.concatenate([z[:half], z[half:]], axis=1)


@functools.cache
def _transpose_fn(N: int, D: int, blk: int):
    grid = (N + blk - 1) // blk
    return pl.pallas_call(
        _transpose_body,
        grid=(grid,),
        in_specs=[
            pl.BlockSpec((D, blk), lambda i: (0, i)),
        ],
        out_specs=pl.BlockSpec((blk // 2, 2 * D), lambda i: (i, 0)),
        out_shape=jax.ShapeDtypeStruct((grid * (blk // 2), 2 * D), jnp.float32),
        compiler_params=pltpu.CompilerParams(vmem_limit_bytes=100 * 2**20),
    )


@functools.cache
def _gather_fn(B: int, N2: int, W: int):
    info = plsc.get_sparse_core_info()
    NC, NS = info.num_cores, info.num_subcores
    NW = NC * NS
    assert B % (NW * IDX_CHUNK) == 0
    n_chunks = B // (NW * IDX_CHUNK)
    b_per_w = n_chunks * IDX_CHUNK
    mesh = plsc.VectorSubcoreMesh(core_axis_name="c", subcore_axis_name="s")

    @functools.partial(
        pl.kernel,
        mesh=mesh,
        out_type=jax.ShapeDtypeStruct((B, W), jnp.float32),
        scratch_types=[
            pltpu.VMEM((n_chunks, IDX_CHUNK), jnp.int32),
            pltpu.VMEM((b_per_w, W), jnp.float32),
            pltpu.SemaphoreType.DMA,
        ],
    )
    def gather(idx_hbm, table_hbm, out_hbm, idx_v, rows_v, sem):
        wid = lax.axis_index("s") * NC + lax.axis_index("c")
        base = wid * b_per_w
        pltpu.sync_copy(idx_hbm.at[wid], idx_v)
        copies = []
        for c in range(n_chunks):
            copies.append(
                pltpu.async_copy(
                    table_hbm.at[idx_v.at[c]],
                    rows_v.at[pl.ds(c * IDX_CHUNK, IDX_CHUNK)],
                    sem,
                )
            )
        for cp in copies:
            cp.wait()
        pltpu.sync_copy(rows_v, out_hbm.at[pl.ds(base, b_per_w)])

    def run(idx_half, table2):
        idx3 = idx_half.reshape(NW, n_chunks, IDX_CHUNK)
        return gather(idx3, table2)

    return run


def _mlp_body(packed_ref, half_ref, w1_ref, b1_ref, w2_ref, b2_ref, out_ref):
    par = half_ref[...].astype(jnp.float32)
    p0 = packed_ref[:, :EMBED_DIM]
    p1 = packed_ref[:, EMBED_DIM:]
    emb = p0 * (1.0 - par) + p1 * par
    h = jnp.dot(emb, w1_ref[...], preferred_element_type=jnp.float32)
    h = jnp.maximum(h + b1_ref[...], 0.0)
    out_ref[...] = (
        jnp.dot(h, w2_ref[...], preferred_element_type=jnp.float32) + b2_ref[...]
    )


@functools.cache
def _mlp_fn(B: int, D: int, H: int, blk: int):
    grid = B // blk
    return pl.pallas_call(
        _mlp_body,
        grid=(grid,),
        in_specs=[
            pl.BlockSpec((blk, 2 * D), lambda i: (i, 0)),
            pl.BlockSpec((blk, 1), lambda i: (i, 0)),
            pl.BlockSpec((D, H), lambda i: (0, 0)),
            pl.BlockSpec((1, H), lambda i: (0, 0)),
            pl.BlockSpec((H, D), lambda i: (0, 0)),
            pl.BlockSpec((1, D), lambda i: (0, 0)),
        ],
        out_specs=pl.BlockSpec((blk, D), lambda i: (i, 0)),
        out_shape=jax.ShapeDtypeStruct((B, D), jnp.float32),
    )


def kernel(user_id, table, W1, b1, W2, b2):
    B = user_id.shape[0]
    N, D = table.shape
    H = W1.shape[1]
    blk = 32768
    idx = user_id.reshape(B).astype(jnp.int32)
    table2 = _transpose_fn(N, D, blk)(table.T)
    blk_i = idx // blk
    off = idx % blk
    packed_row = blk_i * (blk // 2) + off % (blk // 2)
    half = off // (blk // 2)
    packed = _gather_fn(B, table2.shape[0], 2 * D)(packed_row, table2)
    out = _mlp_fn(B, D, H, 2048)(
        packed,
        half.reshape(B, 1),
        W1,
        b1.reshape(1, H),
        W2,
        b2.reshape(1, D),
    )
    return out
